# baseline (speedup 1.0000x reference)
import functools

import jax
import jax.numpy as jnp
from jax.experimental import pallas as pl
from jax.experimental.pallas import tpu as pltpu

_N = 1024
_D = 32
_OPTIONS = 512
_NUM_STAGES = 4


def _encoder_body(x_ref, cb_ref, bias_ref, enc_ref, cur_ref, loss_hbm,
                  loss_scr, sems):
    x = x_ref[...]
    n = x.shape[0]
    current = jnp.zeros_like(x)
    ones_col = jnp.ones((n, 1), jnp.float32)
    ones_row = jnp.ones((1, _OPTIONS), jnp.float32)
    idxs = []
    copies = []
    eye = (
        jax.lax.broadcasted_iota(jnp.int32, (_D, _D), 0)
        == jax.lax.broadcasted_iota(jnp.int32, (_D, _D), 1)
    ).astype(jnp.float32)
    for i in range(_NUM_STAGES):
        cb = cb_ref[i]
        if i == 0:
            cb = cb + bias_ref[...]
        cbt = jax.lax.dot_general(
            eye, cb, (((1,), (1,)), ((), ())),
            preferred_element_type=jnp.float32,
            precision=jax.lax.Precision.HIGHEST,
        )
        sq_c = jnp.sum(cbt * cbt, axis=0, keepdims=True) * (1.0 / _D)
        b_aug = jnp.concatenate([cbt, ones_row, sq_c], axis=0)
        r = current - x
        sq_r = jnp.sum(r * r, axis=1, keepdims=True) * (1.0 / _D)
        a_aug = jnp.concatenate([r * (2.0 / _D), sq_r, ones_col], axis=1)
        loss = jax.lax.dot_general(
            a_aug, b_aug, (((1,), (0,)), ((), ())),
            preferred_element_type=jnp.float32,
            precision=jax.lax.Precision.HIGHEST,
        )
        loss_scr[i] = loss
        cp = pltpu.make_async_copy(
            loss_scr.at[i], loss_hbm.at[:, i, :], sems.at[i]
        )
        cp.start()
        copies.append(cp)
        idx = jnp.argmin(loss, axis=1)
        idxs.append(idx)
        onehot_t = (
            jax.lax.broadcasted_iota(jnp.int32, (_OPTIONS, n), 0)
            == idx[None, :]
        ).astype(jnp.float32)
        chosen_t = jax.lax.dot_general(
            cbt, onehot_t, (((1,), (0,)), ((), ())),
            preferred_element_type=jnp.float32,
            precision=jax.lax.Precision.HIGHEST,
        )
        current = current + chosen_t.T
    enc_ref[...] = jnp.stack(idxs, axis=1)
    cur_ref[...] = current
    for cp in copies:
        cp.wait()


@jax.jit
def kernel(inputs, codebook, bias):
    n, d = inputs.shape
    num_stages, options, _ = codebook.shape
    enc, current, losses = pl.pallas_call(
        _encoder_body,
        in_specs=[
            pl.BlockSpec((n, d), lambda: (0, 0)),
            pl.BlockSpec((num_stages, options, d), lambda: (0, 0, 0)),
            pl.BlockSpec((options, d), lambda: (0, 0)),
        ],
        out_specs=[
            pl.BlockSpec((n, num_stages), lambda: (0, 0)),
            pl.BlockSpec((n, d), lambda: (0, 0)),
            pl.BlockSpec(memory_space=pltpu.MemorySpace.HBM),
        ],
        out_shape=[
            jax.ShapeDtypeStruct((n, num_stages), jnp.int32),
            jax.ShapeDtypeStruct((n, d), jnp.float32),
            jax.ShapeDtypeStruct((n, num_stages, options), jnp.float32),
        ],
        scratch_shapes=[
            pltpu.VMEM((num_stages, n, options), jnp.float32),
            pltpu.SemaphoreType.DMA((num_stages,)),
        ],
    )(inputs, codebook, bias)
    return enc, current, losses


# speedup vs baseline: 1.1378x; 1.1378x over previous
import functools

import jax
import jax.numpy as jnp
from jax.experimental import pallas as pl
from jax.experimental.pallas import tpu as pltpu

_N = 1024
_D = 32
_OPTIONS = 512
_NUM_STAGES = 4


def _encoder_body(x_ref, cbt_ref, bias_ref, enc_ref, cur_ref, loss_hbm,
                  loss_scr, sems):
    x = x_ref[...]
    n = x.shape[0]
    current = jnp.zeros_like(x)
    ones_col = jnp.ones((n, 1), jnp.float32)
    ones_row = jnp.ones((1, _OPTIONS), jnp.float32)
    idxs = []
    copies = []
    for i in range(_NUM_STAGES):
        cbt = cbt_ref[i]
        if i == 0:
            cbt = cbt + bias_ref[...].T
        sq_c = jnp.sum(cbt * cbt, axis=0, keepdims=True) * (1.0 / _D)
        b_aug = jnp.concatenate([cbt, ones_row, sq_c], axis=0)
        r = current - x
        sq_r = jnp.sum(r * r, axis=1, keepdims=True) * (1.0 / _D)
        a_aug = jnp.concatenate([r * (2.0 / _D), sq_r, ones_col], axis=1)
        loss = jax.lax.dot_general(
            a_aug, b_aug, (((1,), (0,)), ((), ())),
            preferred_element_type=jnp.float32,
            precision=jax.lax.Precision.HIGHEST,
        )
        loss_scr[i] = loss
        cp = pltpu.make_async_copy(
            loss_scr.at[i], loss_hbm.at[:, i, :], sems.at[i]
        )
        cp.start()
        copies.append(cp)
        idx = jnp.argmin(loss, axis=1)
        idxs.append(idx)
        onehot_t = (
            jax.lax.broadcasted_iota(jnp.int32, (_OPTIONS, n), 0)
            == idx[None, :]
        ).astype(jnp.float32)
        chosen_t = jax.lax.dot_general(
            cbt, onehot_t, (((1,), (0,)), ((), ())),
            preferred_element_type=jnp.float32,
            precision=jax.lax.Precision.HIGHEST,
        )
        current = current + chosen_t.T
    enc_ref[...] = jnp.stack(idxs, axis=1)
    cur_ref[...] = current
    for cp in copies:
        cp.wait()


@jax.jit
def kernel(inputs, codebook, bias):
    n, d = inputs.shape
    num_stages, options, _ = codebook.shape
    cbt = jnp.swapaxes(codebook, 1, 2)
    enc, current, losses = pl.pallas_call(
        _encoder_body,
        in_specs=[
            pl.BlockSpec((n, d), lambda: (0, 0)),
            pl.BlockSpec((num_stages, d, options), lambda: (0, 0, 0)),
            pl.BlockSpec((options, d), lambda: (0, 0)),
        ],
        out_specs=[
            pl.BlockSpec((n, num_stages), lambda: (0, 0)),
            pl.BlockSpec((n, d), lambda: (0, 0)),
            pl.BlockSpec(memory_space=pltpu.MemorySpace.HBM),
        ],
        out_shape=[
            jax.ShapeDtypeStruct((n, num_stages), jnp.int32),
            jax.ShapeDtypeStruct((n, d), jnp.float32),
            jax.ShapeDtypeStruct((n, num_stages, options), jnp.float32),
        ],
        scratch_shapes=[
            pltpu.VMEM((num_stages, n, options), jnp.float32),
            pltpu.SemaphoreType.DMA((num_stages,)),
        ],
    )(inputs, cbt, bias)
    return enc, current, losses


# speedup vs baseline: 1.1477x; 1.0087x over previous
import functools

import jax
import jax.numpy as jnp
from jax.experimental import pallas as pl
from jax.experimental.pallas import tpu as pltpu

_N = 1024
_D = 32
_OPTIONS = 512
_NUM_STAGES = 4


def _encoder_body(x_ref, cbt_ref, bias_ref, enc_ref, cur_ref, loss_hbm,
                  loss_scr, sems):
    x = x_ref[...]
    n = x.shape[0]
    x_t = x.T
    current_t = jnp.zeros_like(x_t)
    ones_row_n = jnp.ones((1, n), jnp.float32)
    ones_row = jnp.ones((1, _OPTIONS), jnp.float32)
    idxs = []
    copies = []
    for i in range(_NUM_STAGES):
        cbt = cbt_ref[i]
        if i == 0:
            cbt = cbt + bias_ref[...].T
        sq_c = jnp.sum(cbt * cbt, axis=0, keepdims=True) * (1.0 / _D)
        b_aug = jnp.concatenate([cbt, ones_row, sq_c], axis=0)
        r_t = current_t - x_t
        sq_r = jnp.sum(r_t * r_t, axis=0, keepdims=True) * (1.0 / _D)
        a_aug_t = jnp.concatenate([r_t * (2.0 / _D), sq_r, ones_row_n], axis=0)
        loss = jax.lax.dot_general(
            a_aug_t, b_aug, (((0,), (0,)), ((), ())),
            preferred_element_type=jnp.float32,
            precision=jax.lax.Precision.HIGHEST,
        )
        loss_scr[i] = loss
        cp = pltpu.make_async_copy(
            loss_scr.at[i], loss_hbm.at[:, i, :], sems.at[i]
        )
        cp.start()
        copies.append(cp)
        idx = jnp.argmin(loss, axis=1)
        idxs.append(idx)
        onehot_t = (
            jax.lax.broadcasted_iota(jnp.int32, (_OPTIONS, n), 0)
            == idx[None, :]
        ).astype(jnp.float32)
        chosen_t = jax.lax.dot_general(
            cbt, onehot_t, (((1,), (0,)), ((), ())),
            preferred_element_type=jnp.float32,
            precision=jax.lax.Precision.HIGHEST,
        )
        current_t = current_t + chosen_t
    enc_ref[...] = jnp.stack(idxs, axis=1)
    cur_ref[...] = current_t.T
    for cp in copies:
        cp.wait()


@jax.jit
def kernel(inputs, codebook, bias):
    n, d = inputs.shape
    num_stages, options, _ = codebook.shape
    cbt = jnp.swapaxes(codebook, 1, 2)
    enc, current, losses = pl.pallas_call(
        _encoder_body,
        in_specs=[
            pl.BlockSpec((n, d), lambda: (0, 0)),
            pl.BlockSpec((num_stages, d, options), lambda: (0, 0, 0)),
            pl.BlockSpec((options, d), lambda: (0, 0)),
        ],
        out_specs=[
            pl.BlockSpec((n, num_stages), lambda: (0, 0)),
            pl.BlockSpec((n, d), lambda: (0, 0)),
            pl.BlockSpec(memory_space=pltpu.MemorySpace.HBM),
        ],
        out_shape=[
            jax.ShapeDtypeStruct((n, num_stages), jnp.int32),
            jax.ShapeDtypeStruct((n, d), jnp.float32),
            jax.ShapeDtypeStruct((n, num_stages, options), jnp.float32),
        ],
        scratch_shapes=[
            pltpu.VMEM((num_stages, n, options), jnp.float32),
            pltpu.SemaphoreType.DMA((num_stages,)),
        ],
    )(inputs, cbt, bias)
    return enc, current, losses
